# baseline (speedup 1.0000x reference)
import functools

import jax
import jax.numpy as jnp
from jax import lax
from jax.experimental import pallas as pl
from jax.experimental.pallas import tpu as pltpu
from jax.experimental.pallas import tpu_sc as plsc

N = 100000
DIN = 128
DH = 32
NCLS = 2
HALF = 16
NCORE = 2
NSUB = 16
GARB = 352
NPAD = N + GARB
CHUNK = 128
BLK = 3
STRIPE = NPAD // NSUB
ZCH = STRIPE // 4
WB = 7
BN = NPAD // 16

_MESH = plsc.VectorSubcoreMesh(core_axis_name="c", subcore_axis_name="s")
_SC_PARAMS = pltpu.CompilerParams(use_tc_tiling_on_sc=False)


def _writeback(acc, out_hbm, wbx_c, wbv, st, ssems, s):
    @pl.loop(0, WB)
    def _(k):
        pltpu.sync_copy(wbx_c.at[pl.ds(s * (WB * WB) + k * WB, WB)], wbv)
        for i in range(WB):
            b = i % 3
            if i >= 3:
                pltpu.make_async_copy(st[b], out_hbm.at[wbv.at[i - 3]],
                                      ssems[b]).wait()
            pltpu.sync_copy(
                acc.at[pl.ds(s * STRIPE + (k * WB + i) * CHUNK, CHUNK)],
                st[b])
            pltpu.async_copy(st[b], out_hbm.at[wbv.at[i]], ssems[b])
        for i in range(WB - 3, WB):
            b = i % 3
            pltpu.make_async_copy(st[b], out_hbm.at[wbv.at[i]],
                                  ssems[b]).wait()


def _hist_body(dst_hbm, zeros_hbm, ones_hbm, wbx_hbm, out_hbm, onesv,
               didx0, didx1, wbv, st0, st1, st2, acc, sem0, sem1, sem2):
    c = lax.axis_index("c")
    s = lax.axis_index("s")
    base0 = s * STRIPE
    for i in range(4):
        pltpu.sync_copy(zeros_hbm, acc.at[pl.ds(base0 + i * ZCH, ZCH)])
    pltpu.sync_copy(ones_hbm, onesv)
    plsc.subcore_barrier()
    nrows = dst_hbm.shape[0] // (NCORE * NSUB)
    nblk = nrows // BLK
    rbase = (c * NSUB + s) * nrows
    didx = (didx0, didx1)
    sems = (sem0, sem1)

    def load_fire(row, p):
        pltpu.sync_copy(dst_hbm.at[pl.ds(row, BLK)], didx[p])
        for j in range(BLK):
            pltpu.async_copy(onesv, acc.at[didx[p].at[j]], sems[p], add=True)

    def drain(p):
        for j in range(BLK):
            pltpu.make_async_copy(onesv, acc.at[didx[p].at[j]], sems[p]).wait()

    load_fire(rbase, 0)
    load_fire(rbase + BLK, 1)

    @pl.loop(0, (nblk - 3) // 2)
    def _(k):
        for p in range(2):
            drain(p)
            load_fire(rbase + (2 + 2 * k + p) * BLK, p)

    drain(0)
    load_fire(rbase + (nblk - 1) * BLK, 0)
    drain(1)
    drain(0)

    plsc.subcore_barrier()
    _writeback(acc, out_hbm, wbx_hbm.at[c], wbv, (st0, st1, st2),
               (sem0, sem1, sem2), s)


def _sc_hist(dst2, zeros_blk, ones_blk, wbx):
    k = pl.kernel(
        _hist_body,
        mesh=_MESH,
        out_type=jax.ShapeDtypeStruct((8 * NPAD, HALF), jnp.float32),
        scratch_types=[
            pltpu.VMEM((CHUNK, HALF), jnp.float32),
            pltpu.VMEM((BLK, CHUNK), jnp.int32),
            pltpu.VMEM((BLK, CHUNK), jnp.int32),
            pltpu.VMEM((WB, CHUNK), jnp.int32),
            pltpu.VMEM((CHUNK, HALF), jnp.float32),
            pltpu.VMEM((CHUNK, HALF), jnp.float32),
            pltpu.VMEM((CHUNK, HALF), jnp.float32),
            pltpu.VMEM_SHARED((NPAD, HALF), jnp.float32),
            pltpu.SemaphoreType.DMA,
            pltpu.SemaphoreType.DMA,
            pltpu.SemaphoreType.DMA,
        ],
        compiler_params=_SC_PARAMS,
    )
    return k(dst2, zeros_blk, ones_blk, wbx)


def _conv_body(hs_hbm, sd_hbm, zeros_hbm, wbx_hbm, out_hbm, *scr):
    sd = scr[0:4]
    rows = scr[4:8]
    wbv = scr[8]
    acc = scr[9]
    semg = scr[10:14]
    sems = scr[14:18]
    c = lax.axis_index("c")
    s = lax.axis_index("s")
    base0 = s * STRIPE
    for i in range(4):
        pltpu.sync_copy(zeros_hbm, acc.at[pl.ds(base0 + i * ZCH, ZCH)])
    plsc.subcore_barrier()
    nrows = sd_hbm.shape[1] // NSUB
    nblk = nrows // (2 * BLK)
    rbase = s * nrows
    sd_c = sd_hbm.at[c]

    def load_fire_gathers(b, q):
        pltpu.sync_copy(sd_c.at[pl.ds(rbase + b * 2 * BLK, 2 * BLK)], sd[q])
        for j in range(BLK):
            pltpu.async_copy(hs_hbm.at[sd[q].at[j]], rows[q].at[j], semg[q])

    def wait_gathers(q):
        for j in range(BLK):
            pltpu.make_async_copy(hs_hbm.at[sd[q].at[j]], rows[q].at[j],
                                  semg[q]).wait()

    def fire_scatters(q):
        for j in range(BLK):
            pltpu.async_copy(rows[q].at[j], acc.at[sd[q].at[BLK + j]], sems[q],
                             add=True)

    def drain_scatters(q):
        for j in range(BLK):
            pltpu.make_async_copy(rows[q].at[j], acc.at[sd[q].at[BLK + j]],
                                  sems[q]).wait()

    load_fire_gathers(0, 0)
    load_fire_gathers(1, 1)

    for p in range(4):
        wait_gathers(p)
        fire_scatters(p)
        q = (p + 2) % 4
        if p >= 2:
            drain_scatters(q)
        load_fire_gathers(p + 2, q)

    @pl.loop(1, (nblk - 2) // 4)
    def _(k):
        for p in range(4):
            wait_gathers(p)
            fire_scatters(p)
            q = (p + 2) % 4
            drain_scatters(q)
            load_fire_gathers(4 * k + p + 2, q)

    wait_gathers(0)
    fire_scatters(0)
    drain_scatters(2)
    wait_gathers(1)
    fire_scatters(1)
    drain_scatters(3)
    drain_scatters(0)
    drain_scatters(1)

    plsc.subcore_barrier()
    _writeback(acc, out_hbm, wbx_hbm.at[c], wbv,
               tuple(rows[0].at[i] for i in range(3)), semg[0:3], s)


def _sc_conv(hs_view, sdx, zeros_blk, wbx):
    k = pl.kernel(
        _conv_body,
        mesh=_MESH,
        out_type=jax.ShapeDtypeStruct((8 * NPAD, HALF), jnp.float32),
        scratch_types=(
            [pltpu.VMEM((2 * BLK, CHUNK), jnp.int32) for _ in range(4)]
            + [pltpu.VMEM((BLK, CHUNK, HALF), jnp.float32)
               for _ in range(4)]
            + [pltpu.VMEM((WB, CHUNK), jnp.int32)]
            + [pltpu.VMEM_SHARED((NPAD, HALF), jnp.float32)]
            + [pltpu.SemaphoreType.DMA for _ in range(8)]
        ),
        compiler_params=_SC_PARAMS,
    )
    return k(hs_view, sdx, zeros_blk, wbx)


def _dense_in_body(x_ref, wi_ref, bi_ref, wg_ref, g1_ref):
    h0 = jnp.dot(x_ref[...], wi_ref[...],
                 preferred_element_type=jnp.float32) + bi_ref[...]
    h0 = jnp.where(h0 > 0, h0, 0.01 * h0)
    g1_ref[:, 0:DH] = jnp.dot(h0, wg_ref[...],
                              preferred_element_type=jnp.float32)


def _dense_in(x, w_in, b_in2, w_gcn):
    return pl.pallas_call(
        _dense_in_body,
        grid=(NPAD // BN,),
        in_specs=[
            pl.BlockSpec((BN, DIN), lambda i: (i, 0)),
            pl.BlockSpec((DIN, DH), lambda i: (0, 0)),
            pl.BlockSpec((1, DH), lambda i: (0, 0)),
            pl.BlockSpec((DH, DH), lambda i: (0, 0)),
        ],
        out_specs=pl.BlockSpec((BN, 128), lambda i: (i, 0)),
        out_shape=jax.ShapeDtypeStruct((NPAD, 128), jnp.float32),
    )(x, w_in, b_in2, w_gcn)


def _scale_body(cnt_ref, g1_ref, hsd_ref):
    deg = 1.0 + cnt_ref[:, 0:HALF] + cnt_ref[:, HALF:DH]
    dinv = lax.rsqrt(deg)
    dinv32 = jnp.concatenate([dinv, dinv], axis=1)
    hsd_ref[:, 0:DH] = dinv32 * g1_ref[:, 0:DH]
    hsd_ref[:, DH:2 * DH] = dinv32


def _scale(counts, g1):
    return pl.pallas_call(
        _scale_body,
        grid=(NPAD // BN,),
        in_specs=[
            pl.BlockSpec((BN, 128), lambda i: (i, 0)),
            pl.BlockSpec((BN, 128), lambda i: (i, 0)),
        ],
        out_specs=pl.BlockSpec((BN, 128), lambda i: (i, 0)),
        out_shape=jax.ShapeDtypeStruct((NPAD, 128), jnp.float32),
    )(counts, g1)


def _mid_body(agg_ref, hsd_ref, wg_ref, bg_ref, hs2_ref):
    dinv32 = hsd_ref[:, DH:2 * DH]
    h1 = dinv32 * (agg_ref[:, 0:DH] + hsd_ref[:, 0:DH]) + bg_ref[...]
    g2 = jnp.dot(h1, wg_ref[...], preferred_element_type=jnp.float32)
    hs2_ref[:, 0:DH] = dinv32 * g2


def _mid(agg1, hsd, w_gcn, bg2):
    return pl.pallas_call(
        _mid_body,
        grid=(NPAD // BN,),
        in_specs=[
            pl.BlockSpec((BN, 128), lambda i: (i, 0)),
            pl.BlockSpec((BN, 128), lambda i: (i, 0)),
            pl.BlockSpec((DH, DH), lambda i: (0, 0)),
            pl.BlockSpec((1, DH), lambda i: (0, 0)),
        ],
        out_specs=pl.BlockSpec((BN, 128), lambda i: (i, 0)),
        out_shape=jax.ShapeDtypeStruct((NPAD, 128), jnp.float32),
    )(agg1, hsd, w_gcn, bg2)


def _head_body(agg_ref, hs2_ref, hsd_ref, bg_ref, w1_ref, b1_ref,
               w2_ref, b2_ref, out_ref):
    dinv32 = hsd_ref[:, DH:2 * DH]
    h2 = dinv32 * (agg_ref[:, 0:DH] + hs2_ref[:, 0:DH]) + bg_ref[...]
    t = jnp.dot(h2, w1_ref[...], preferred_element_type=jnp.float32) + b1_ref[...]
    t = jnp.where(t > 0, t, 0.01 * t)
    out_ref[...] = jnp.dot(t, w2_ref[...],
                           preferred_element_type=jnp.float32) + b2_ref[...]


def _head(agg2, hs2, hsd, bg2, w1, b12, w2, b22):
    return pl.pallas_call(
        _head_body,
        grid=(NPAD // BN,),
        in_specs=[
            pl.BlockSpec((BN, 128), lambda i: (i, 0)),
            pl.BlockSpec((BN, 128), lambda i: (i, 0)),
            pl.BlockSpec((BN, 128), lambda i: (i, 0)),
            pl.BlockSpec((1, DH), lambda i: (0, 0)),
            pl.BlockSpec((DH, DH), lambda i: (0, 0)),
            pl.BlockSpec((1, DH), lambda i: (0, 0)),
            pl.BlockSpec((DH, NCLS), lambda i: (0, 0)),
            pl.BlockSpec((1, NCLS), lambda i: (0, 0)),
        ],
        out_specs=pl.BlockSpec((BN, NCLS), lambda i: (i, 0)),
        out_shape=jax.ShapeDtypeStruct((N, NCLS), jnp.float32),
    )(agg2, hs2, hsd, bg2, w1, b12, w2, b22)


def kernel(x, edge_index, edge_type, W_in, b_in, W_gcn, b_gcn, W1, b1, W2, b2):
    del edge_type
    e = edge_index.shape[1]
    nblk = -(-e // (NSUB * BLK * CHUNK))
    nblk += (2 - nblk) % 4
    epad = NSUB * BLK * CHUNK * nblk
    npad_e = epad - e
    ar = jnp.arange(npad_e, dtype=jnp.int32)
    src = jnp.concatenate([edge_index[0], ar % N])
    dst = jnp.concatenate([edge_index[1], N + (ar % GARB)])
    src8 = src * 8
    srcx = jnp.stack([src8, src8 + 1]).reshape(NCORE, -1, BLK, CHUNK)
    dst2 = dst.reshape(epad // CHUNK, CHUNK)
    dstb = jnp.broadcast_to(dst2.reshape(1, -1, BLK, CHUNK), srcx.shape)
    sdx = jnp.concatenate([srcx, dstb], axis=2).reshape(NCORE, -1, CHUNK)
    wb = jnp.arange(NPAD, dtype=jnp.int32) * 8
    wbx = jnp.stack([wb, wb + 1]).reshape(NCORE, NPAD // CHUNK, CHUNK)
    zeros_blk = jnp.zeros((ZCH, HALF), jnp.float32)
    ones_blk = jnp.ones((CHUNK, HALF), jnp.float32)
    b_in2 = b_in.reshape(1, DH)
    bg2 = b_gcn.reshape(1, DH)
    b12 = b1.reshape(1, DH)
    b22 = b2.reshape(1, NCLS)

    counts = _sc_hist(dst2, zeros_blk, ones_blk, wbx).reshape(NPAD, 128)
    g1 = _dense_in(x, W_in, b_in2, W_gcn)
    hsd = _scale(counts, g1)
    agg1 = _sc_conv(hsd.reshape(8 * NPAD, HALF), sdx, zeros_blk,
                    wbx).reshape(NPAD, 128)
    hs2 = _mid(agg1, hsd, W_gcn, bg2)
    agg2 = _sc_conv(hs2.reshape(8 * NPAD, HALF), sdx, zeros_blk,
                    wbx).reshape(NPAD, 128)
    return _head(agg2, hs2, hsd, bg2, W1, b12, W2, b22)


# speedup vs baseline: 1.0199x; 1.0199x over previous
import functools

import jax
import jax.numpy as jnp
from jax import lax
from jax.experimental import pallas as pl
from jax.experimental.pallas import tpu as pltpu
from jax.experimental.pallas import tpu_sc as plsc

N = 100000
DIN = 128
DH = 32
NCLS = 2
HALF = 16
NCORE = 2
NSUB = 16
GARB = 352
NPAD = N + GARB
CHUNK = 128
BLK = 3
STRIPE = NPAD // NSUB
ZCH = STRIPE // 4
WB = 7
BN = NPAD // 16

_MESH = plsc.VectorSubcoreMesh(core_axis_name="c", subcore_axis_name="s")
_SC_PARAMS = pltpu.CompilerParams(use_tc_tiling_on_sc=False)


def _writeback(acc, out_hbm, wbx_c, wbv, st, ssems, s):
    @pl.loop(0, WB)
    def _(k):
        pltpu.sync_copy(wbx_c.at[pl.ds(s * (WB * WB) + k * WB, WB)], wbv)
        for i in range(WB):
            b = i % 3
            if i >= 3:
                pltpu.make_async_copy(st[b], out_hbm.at[wbv.at[i - 3]],
                                      ssems[b]).wait()
            pltpu.sync_copy(
                acc.at[pl.ds(s * STRIPE + (k * WB + i) * CHUNK, CHUNK)],
                st[b])
            pltpu.async_copy(st[b], out_hbm.at[wbv.at[i]], ssems[b])
        for i in range(WB - 3, WB):
            b = i % 3
            pltpu.make_async_copy(st[b], out_hbm.at[wbv.at[i]],
                                  ssems[b]).wait()


def _hist_body(dst_hbm, zeros_hbm, ones_hbm, wbx_hbm, out_hbm, onesv,
               didx0, didx1, wbv, st0, st1, st2, acc, sem0, sem1, sem2):
    c = lax.axis_index("c")
    s = lax.axis_index("s")
    base0 = s * STRIPE
    for i in range(4):
        pltpu.sync_copy(zeros_hbm, acc.at[pl.ds(base0 + i * ZCH, ZCH)])
    pltpu.sync_copy(ones_hbm, onesv)
    plsc.subcore_barrier()
    nrows = dst_hbm.shape[0] // (NCORE * NSUB)
    nblk = nrows // BLK
    rbase = (c * NSUB + s) * nrows
    didx = (didx0, didx1)
    sems = (sem0, sem1)

    def load_fire(row, p):
        pltpu.sync_copy(dst_hbm.at[pl.ds(row, BLK)], didx[p])
        for j in range(BLK):
            pltpu.async_copy(onesv, acc.at[didx[p].at[j]], sems[p], add=True)

    def drain(p):
        for j in range(BLK):
            pltpu.make_async_copy(onesv, acc.at[didx[p].at[j]], sems[p]).wait()

    load_fire(rbase, 0)
    load_fire(rbase + BLK, 1)

    @pl.loop(0, (nblk - 3) // 2)
    def _(k):
        for p in range(2):
            drain(p)
            load_fire(rbase + (2 + 2 * k + p) * BLK, p)

    drain(0)
    load_fire(rbase + (nblk - 1) * BLK, 0)
    drain(1)
    drain(0)

    plsc.subcore_barrier()
    _writeback(acc, out_hbm, wbx_hbm.at[c], wbv, (st0, st1, st2),
               (sem0, sem1, sem2), s)


def _sc_hist(dst2, zeros_blk, ones_blk, wbx):
    k = pl.kernel(
        _hist_body,
        mesh=_MESH,
        out_type=jax.ShapeDtypeStruct((8 * NPAD, HALF), jnp.float32),
        scratch_types=[
            pltpu.VMEM((CHUNK, HALF), jnp.float32),
            pltpu.VMEM((BLK, CHUNK), jnp.int32),
            pltpu.VMEM((BLK, CHUNK), jnp.int32),
            pltpu.VMEM((WB, CHUNK), jnp.int32),
            pltpu.VMEM((CHUNK, HALF), jnp.float32),
            pltpu.VMEM((CHUNK, HALF), jnp.float32),
            pltpu.VMEM((CHUNK, HALF), jnp.float32),
            pltpu.VMEM_SHARED((NPAD, HALF), jnp.float32),
            pltpu.SemaphoreType.DMA,
            pltpu.SemaphoreType.DMA,
            pltpu.SemaphoreType.DMA,
        ],
        compiler_params=_SC_PARAMS,
    )
    return k(dst2, zeros_blk, ones_blk, wbx)


def _conv_body(hs_hbm, src_hbm, dst_hbm, zeros_hbm, wbx_hbm, out_hbm, *scr):
    sidx = scr[0:4]
    didx = scr[4:8]
    rows = scr[8:12]
    wbv = scr[12]
    acc = scr[13]
    semg = scr[14:18]
    sems = scr[18:22]
    c = lax.axis_index("c")
    s = lax.axis_index("s")
    base0 = s * STRIPE
    for i in range(4):
        pltpu.sync_copy(zeros_hbm, acc.at[pl.ds(base0 + i * ZCH, ZCH)])
    plsc.subcore_barrier()
    nrows = src_hbm.shape[1] // NSUB
    nblk = nrows // BLK
    rbase = s * nrows
    src_c = src_hbm.at[c]

    def load_fire_gathers(b, q):
        row = rbase + b * BLK
        pltpu.sync_copy(src_c.at[pl.ds(row, BLK)], sidx[q])
        pltpu.sync_copy(dst_hbm.at[pl.ds(row, BLK)], didx[q])
        for j in range(BLK):
            pltpu.async_copy(hs_hbm.at[sidx[q].at[j]], rows[q].at[j], semg[q])

    def wait_gathers(q):
        for j in range(BLK):
            pltpu.make_async_copy(hs_hbm.at[sidx[q].at[j]], rows[q].at[j],
                                  semg[q]).wait()

    def fire_scatters(q):
        for j in range(BLK):
            pltpu.async_copy(rows[q].at[j], acc.at[didx[q].at[j]], sems[q],
                             add=True)

    def drain_scatters(q):
        for j in range(BLK):
            pltpu.make_async_copy(rows[q].at[j], acc.at[didx[q].at[j]],
                                  sems[q]).wait()

    load_fire_gathers(0, 0)
    load_fire_gathers(1, 1)

    for p in range(4):
        wait_gathers(p)
        fire_scatters(p)
        q = (p + 2) % 4
        if p >= 2:
            drain_scatters(q)
        load_fire_gathers(p + 2, q)

    @pl.loop(1, (nblk - 2) // 4)
    def _(k):
        for p in range(4):
            wait_gathers(p)
            fire_scatters(p)
            q = (p + 2) % 4
            drain_scatters(q)
            load_fire_gathers(4 * k + p + 2, q)

    wait_gathers(0)
    fire_scatters(0)
    drain_scatters(2)
    wait_gathers(1)
    fire_scatters(1)
    drain_scatters(3)
    drain_scatters(0)
    drain_scatters(1)

    plsc.subcore_barrier()
    _writeback(acc, out_hbm, wbx_hbm.at[c], wbv,
               tuple(rows[0].at[i] for i in range(3)), semg[0:3], s)


def _sc_conv(hs_view, srcx, dst2, zeros_blk, wbx):
    k = pl.kernel(
        _conv_body,
        mesh=_MESH,
        out_type=jax.ShapeDtypeStruct((8 * NPAD, HALF), jnp.float32),
        scratch_types=(
            [pltpu.VMEM((BLK, CHUNK), jnp.int32) for _ in range(4)]
            + [pltpu.VMEM((BLK, CHUNK), jnp.int32) for _ in range(4)]
            + [pltpu.VMEM((BLK, CHUNK, HALF), jnp.float32)
               for _ in range(4)]
            + [pltpu.VMEM((WB, CHUNK), jnp.int32)]
            + [pltpu.VMEM_SHARED((NPAD, HALF), jnp.float32)]
            + [pltpu.SemaphoreType.DMA for _ in range(8)]
        ),
        compiler_params=_SC_PARAMS,
    )
    return k(hs_view, srcx, dst2, zeros_blk, wbx)


def _dense_in_body(x_ref, wi_ref, bi_ref, wg_ref, g1_ref):
    h0 = jnp.dot(x_ref[...], wi_ref[...],
                 preferred_element_type=jnp.float32) + bi_ref[...]
    h0 = jnp.where(h0 > 0, h0, 0.01 * h0)
    g1_ref[:, 0:DH] = jnp.dot(h0, wg_ref[...],
                              preferred_element_type=jnp.float32)


def _dense_in(x, w_in, b_in2, w_gcn):
    return pl.pallas_call(
        _dense_in_body,
        grid=(NPAD // BN,),
        in_specs=[
            pl.BlockSpec((BN, DIN), lambda i: (i, 0)),
            pl.BlockSpec((DIN, DH), lambda i: (0, 0)),
            pl.BlockSpec((1, DH), lambda i: (0, 0)),
            pl.BlockSpec((DH, DH), lambda i: (0, 0)),
        ],
        out_specs=pl.BlockSpec((BN, 128), lambda i: (i, 0)),
        out_shape=jax.ShapeDtypeStruct((NPAD, 128), jnp.float32),
    )(x, w_in, b_in2, w_gcn)


def _scale_body(cnt_ref, g1_ref, hsd_ref):
    deg = 1.0 + cnt_ref[:, 0:HALF] + cnt_ref[:, HALF:DH]
    dinv = lax.rsqrt(deg)
    dinv32 = jnp.concatenate([dinv, dinv], axis=1)
    hsd_ref[:, 0:DH] = dinv32 * g1_ref[:, 0:DH]
    hsd_ref[:, DH:2 * DH] = dinv32


def _scale(counts, g1):
    return pl.pallas_call(
        _scale_body,
        grid=(NPAD // BN,),
        in_specs=[
            pl.BlockSpec((BN, 128), lambda i: (i, 0)),
            pl.BlockSpec((BN, 128), lambda i: (i, 0)),
        ],
        out_specs=pl.BlockSpec((BN, 128), lambda i: (i, 0)),
        out_shape=jax.ShapeDtypeStruct((NPAD, 128), jnp.float32),
    )(counts, g1)


def _mid_body(agg_ref, hsd_ref, wg_ref, bg_ref, hs2_ref):
    dinv32 = hsd_ref[:, DH:2 * DH]
    h1 = dinv32 * (agg_ref[:, 0:DH] + hsd_ref[:, 0:DH]) + bg_ref[...]
    g2 = jnp.dot(h1, wg_ref[...], preferred_element_type=jnp.float32)
    hs2_ref[:, 0:DH] = dinv32 * g2


def _mid(agg1, hsd, w_gcn, bg2):
    return pl.pallas_call(
        _mid_body,
        grid=(NPAD // BN,),
        in_specs=[
            pl.BlockSpec((BN, 128), lambda i: (i, 0)),
            pl.BlockSpec((BN, 128), lambda i: (i, 0)),
            pl.BlockSpec((DH, DH), lambda i: (0, 0)),
            pl.BlockSpec((1, DH), lambda i: (0, 0)),
        ],
        out_specs=pl.BlockSpec((BN, 128), lambda i: (i, 0)),
        out_shape=jax.ShapeDtypeStruct((NPAD, 128), jnp.float32),
    )(agg1, hsd, w_gcn, bg2)


def _head_body(agg_ref, hs2_ref, hsd_ref, bg_ref, w1_ref, b1_ref,
               w2_ref, b2_ref, out_ref):
    dinv32 = hsd_ref[:, DH:2 * DH]
    h2 = dinv32 * (agg_ref[:, 0:DH] + hs2_ref[:, 0:DH]) + bg_ref[...]
    t = jnp.dot(h2, w1_ref[...], preferred_element_type=jnp.float32) + b1_ref[...]
    t = jnp.where(t > 0, t, 0.01 * t)
    out_ref[...] = jnp.dot(t, w2_ref[...],
                           preferred_element_type=jnp.float32) + b2_ref[...]


def _head(agg2, hs2, hsd, bg2, w1, b12, w2, b22):
    return pl.pallas_call(
        _head_body,
        grid=(NPAD // BN,),
        in_specs=[
            pl.BlockSpec((BN, 128), lambda i: (i, 0)),
            pl.BlockSpec((BN, 128), lambda i: (i, 0)),
            pl.BlockSpec((BN, 128), lambda i: (i, 0)),
            pl.BlockSpec((1, DH), lambda i: (0, 0)),
            pl.BlockSpec((DH, DH), lambda i: (0, 0)),
            pl.BlockSpec((1, DH), lambda i: (0, 0)),
            pl.BlockSpec((DH, NCLS), lambda i: (0, 0)),
            pl.BlockSpec((1, NCLS), lambda i: (0, 0)),
        ],
        out_specs=pl.BlockSpec((BN, NCLS), lambda i: (i, 0)),
        out_shape=jax.ShapeDtypeStruct((N, NCLS), jnp.float32),
    )(agg2, hs2, hsd, bg2, w1, b12, w2, b22)


def kernel(x, edge_index, edge_type, W_in, b_in, W_gcn, b_gcn, W1, b1, W2, b2):
    del edge_type
    e = edge_index.shape[1]
    nblk = -(-e // (NSUB * BLK * CHUNK))
    nblk += (2 - nblk) % 4
    epad = NSUB * BLK * CHUNK * nblk
    npad_e = epad - e
    ar = jnp.arange(npad_e, dtype=jnp.int32)
    src = jnp.concatenate([edge_index[0], ar % N])
    dst = jnp.concatenate([edge_index[1], N + (ar % GARB)])
    src8 = src * 8
    srcx = jnp.stack([src8, src8 + 1]).reshape(NCORE, epad // CHUNK, CHUNK)
    dst2 = dst.reshape(epad // CHUNK, CHUNK)
    wb = jnp.arange(NPAD, dtype=jnp.int32) * 8
    wbx = jnp.stack([wb, wb + 1]).reshape(NCORE, NPAD // CHUNK, CHUNK)
    zeros_blk = jnp.zeros((ZCH, HALF), jnp.float32)
    ones_blk = jnp.ones((CHUNK, HALF), jnp.float32)
    b_in2 = b_in.reshape(1, DH)
    bg2 = b_gcn.reshape(1, DH)
    b12 = b1.reshape(1, DH)
    b22 = b2.reshape(1, NCLS)

    counts = _sc_hist(dst2, zeros_blk, ones_blk, wbx).reshape(NPAD, 128)
    g1 = _dense_in(x, W_in, b_in2, W_gcn)
    hsd = _scale(counts, g1)
    agg1 = _sc_conv(hsd.reshape(8 * NPAD, HALF), srcx, dst2, zeros_blk,
                    wbx).reshape(NPAD, 128)
    hs2 = _mid(agg1, hsd, W_gcn, bg2)
    agg2 = _sc_conv(hs2.reshape(8 * NPAD, HALF), srcx, dst2, zeros_blk,
                    wbx).reshape(NPAD, 128)
    return _head(agg2, hs2, hsd, bg2, W1, b12, W2, b22)
